# Initial kernel scaffold; baseline (speedup 1.0000x reference)
import jax
import jax.numpy as jnp
from jax.experimental import pallas as pl


def kernel(x, edge_index, batch, W1, b1, W2, b2, W3, b3):
    raise NotImplementedError("write your pallas kernel here")


# speedup vs baseline: 9.7918x; 9.7918x over previous
import functools

import jax
import jax.numpy as jnp
from jax import lax
from jax.experimental import pallas as pl
from jax.experimental.pallas import tpu as pltpu
from jax.experimental.pallas import tpu_sc as plsc

N = 10000
E = 320000
D = 128
G = 64

NC, NS, L = 2, 16, 16
NW = NC * NS
EPW = E // NW
CHK = 80
NCHK = EPW // CHK
RPT = 632
RPT_LAST = N - (NS - 1) * RPT


def _tile_rows(s, io_fn):
    @pl.when(s < NS - 1)
    def _():
        io_fn(pl.ds(s * RPT, RPT), RPT)

    @pl.when(s == NS - 1)
    def _():
        io_fn(pl.ds((NS - 1) * RPT, RPT_LAST), RPT_LAST)

_mesh = plsc.VectorSubcoreMesh(
    core_axis_name="c", subcore_axis_name="s", num_cores=NC, num_subcores=NS)


@functools.partial(
    pl.kernel,
    out_type=jax.ShapeDtypeStruct((NW * N,), jnp.float32),
    mesh=_mesh,
    compiler_params=pltpu.CompilerParams(needs_layout_passes=False),
    scratch_types=[
        pltpu.VMEM((EPW,), jnp.int32),
        pltpu.VMEM((N,), jnp.float32),
    ],
)
def _deg_kernel(col_hbm, out_hbm, cidx_v, deg_v):
    c = lax.axis_index("c")
    s = lax.axis_index("s")
    wid = c * NS + s
    pltpu.sync_copy(col_hbm.at[pl.ds(wid * EPW, EPW)], cidx_v)

    def zbody(i, carry):
        deg_v[pl.ds(i * L, L)] = jnp.zeros((L,), jnp.float32)
        return carry

    lax.fori_loop(0, N // L, zbody, 0)
    ones = jnp.ones((L,), jnp.float32)

    def body(i, carry):
        idx = cidx_v[pl.ds(i * L, L)]
        plsc.addupdate_scatter(deg_v, [idx], ones)
        return carry

    lax.fori_loop(0, EPW // L, body, 0)
    pltpu.sync_copy(deg_v, out_hbm.at[pl.ds(wid * N, N)])


@functools.partial(
    pl.kernel,
    out_type=jax.ShapeDtypeStruct((NC, N, D), jnp.float32),
    mesh=_mesh,
    scratch_types=[
        pltpu.VMEM((CHK,), jnp.int32),
        pltpu.VMEM((CHK,), jnp.int32),
        pltpu.VMEM((CHK, D), jnp.float32),
        pltpu.VMEM_SHARED((N, D), jnp.float32),
        pltpu.SemaphoreType.DMA,
    ],
)
def _agg_kernel(g_hbm, row_hbm, col_hbm, z_hbm, out_hbm,
                ridx_v, cidx_v, rows_v, acc_sh, sem):
    c = lax.axis_index("c")
    s = lax.axis_index("s")
    wid = c * NS + s
    _tile_rows(s, lambda sl, cnt: pltpu.sync_copy(
        z_hbm.at[pl.ds(0, cnt)], acc_sh.at[sl]))
    plsc.subcore_barrier()

    def body(k, carry):
        base = wid * EPW + k * CHK
        pltpu.sync_copy(row_hbm.at[pl.ds(base, CHK)], ridx_v)
        pltpu.sync_copy(col_hbm.at[pl.ds(base, CHK)], cidx_v)
        pltpu.async_copy(g_hbm.at[ridx_v], rows_v, sem).wait()
        pltpu.sync_copy(rows_v, acc_sh.at[cidx_v], add=True)
        return carry

    lax.fori_loop(0, NCHK, body, 0)
    plsc.subcore_barrier()
    _tile_rows(s, lambda sl, cnt: pltpu.sync_copy(
        acc_sh.at[sl], out_hbm.at[c, sl]))


@functools.partial(
    pl.kernel,
    out_type=jax.ShapeDtypeStruct((G, D), jnp.float32),
    mesh=_mesh,
    compiler_params=pltpu.CompilerParams(needs_layout_passes=False),
    scratch_types=[
        pltpu.VMEM((N,), jnp.int32),
        pltpu.VMEM((G,), jnp.int32),
        pltpu.VMEM((G, D), jnp.float32),
        pltpu.SemaphoreType.DMA,
    ],
)
def _final_kernel(h3_hbm, batch_hbm, out_hbm, batch_v, idx_v, rows_v, sem):
    c = lax.axis_index("c")
    s = lax.axis_index("s")
    wid = c * NS + s

    @pl.when(wid == 0)
    def _():
        pltpu.sync_copy(batch_hbm, batch_v)
        zero16 = jnp.zeros((L,), jnp.int32)
        for j in range(G // L):
            idx_v[pl.ds(j * L, L)] = zero16

        def body(i, carry):
            cnt, lastv = carry
            lane = lax.iota(jnp.int32, L)
            gpos = lane + i * L
            b = batch_v[pl.ds(i * L, L)]
            prev = b.at[jnp.maximum(lane - 1, 0)].get(mode="promise_in_bounds")
            prev = jnp.where(lane == 0, lastv, prev)
            mask = b != prev
            mi = mask.astype(jnp.int32)
            cum = plsc.cumsum(mi)
            pos = cnt + cum - mi
            wmask = mask & (pos < G)
            plsc.store_scatter(idx_v, [pos], gpos, mask=wmask)
            return cnt + jnp.sum(mi), jnp.max(b)

        lax.fori_loop(0, N // L, body, (0, -1))
        pltpu.async_copy(h3_hbm.at[idx_v], rows_v, sem).wait()
        pltpu.sync_copy(rows_v, out_hbm)


BR = 1000


def _dis_from(degp):
    deg = jnp.sum(degp, axis=1)
    return jnp.where(deg > 0, lax.rsqrt(deg), 0.0)


def _tc1_body(x_ref, w_ref, degp_ref, g_ref):
    dis = _dis_from(degp_ref[...])
    g_ref[...] = jnp.dot(x_ref[...] * dis[:, None], w_ref[...],
                         preferred_element_type=jnp.float32)


def _tc_mid_body(sp_ref, degp_ref, b_ref, w_ref, g_ref):
    dis = _dis_from(degp_ref[...])
    a = dis[:, None] * (sp_ref[0] + sp_ref[1]) + b_ref[...]
    a = jnp.where(a >= 0, a, 0.01 * a)
    g_ref[...] = jnp.dot(a * dis[:, None], w_ref[...],
                         preferred_element_type=jnp.float32)


def _tc_h3_body(sp_ref, degp_ref, b_ref, h_ref):
    dis = _dis_from(degp_ref[...])
    h_ref[...] = dis[:, None] * (sp_ref[0] + sp_ref[1]) + b_ref[...]


_degp_spec = pl.BlockSpec((BR, NW), lambda i: (i, 0))
_row_spec = pl.BlockSpec((BR, D), lambda i: (i, 0))
_w_spec = pl.BlockSpec((D, D), lambda i: (0, 0))
_b_spec = pl.BlockSpec((1, D), lambda i: (0, 0))
_sp_spec = pl.BlockSpec((NC, BR, D), lambda i: (0, i, 0))
_nd_shape = jax.ShapeDtypeStruct((N, D), jnp.float32)

_tc1 = pl.pallas_call(
    _tc1_body, grid=(N // BR,),
    in_specs=[_row_spec, _w_spec, _degp_spec],
    out_specs=_row_spec, out_shape=_nd_shape)

_tc_mid = pl.pallas_call(
    _tc_mid_body, grid=(N // BR,),
    in_specs=[_sp_spec, _degp_spec, _b_spec, _w_spec],
    out_specs=_row_spec, out_shape=_nd_shape)

_tc_h3 = pl.pallas_call(
    _tc_h3_body, grid=(N // BR,),
    in_specs=[_sp_spec, _degp_spec, _b_spec],
    out_specs=_row_spec, out_shape=_nd_shape)


def kernel(x, edge_index, batch, W1, b1, W2, b2, W3, b3):
    row = edge_index[0]
    col = edge_index[1]
    zD = jnp.zeros((RPT, D), jnp.float32)

    degp = _deg_kernel(col).reshape(NW, N).T
    g1 = _tc1(x, W1, degp)
    s1 = _agg_kernel(g1, row, col, zD)
    g2 = _tc_mid(s1, degp, b1.reshape(1, D), W2)
    s2 = _agg_kernel(g2, row, col, zD)
    g3 = _tc_mid(s2, degp, b2.reshape(1, D), W3)
    s3 = _agg_kernel(g3, row, col, zD)
    h3 = _tc_h3(s3, degp, b3.reshape(1, D))
    return _final_kernel(h3, batch)


# speedup vs baseline: 24.7650x; 2.5292x over previous
import functools

import jax
import jax.numpy as jnp
from jax import lax
from jax.experimental import pallas as pl
from jax.experimental.pallas import tpu as pltpu
from jax.experimental.pallas import tpu_sc as plsc

N = 10000
E = 320000
D = 128
G = 64

NC, NS, L = 2, 16, 16
NW = NC * NS
EPW = E // NW
CHK = 64
NCHK = EPW // CHK
CHK_T = EPW - NCHK * CHK
RPT = 632
RPT_LAST = N - (NS - 1) * RPT


def _tile_rows(s, io_fn):
    @pl.when(s < NS - 1)
    def _():
        io_fn(pl.ds(s * RPT, RPT), RPT)

    @pl.when(s == NS - 1)
    def _():
        io_fn(pl.ds((NS - 1) * RPT, RPT_LAST), RPT_LAST)

_mesh = plsc.VectorSubcoreMesh(
    core_axis_name="c", subcore_axis_name="s", num_cores=NC, num_subcores=NS)


@functools.partial(
    pl.kernel,
    out_type=jax.ShapeDtypeStruct((NW * N,), jnp.float32),
    mesh=_mesh,
    compiler_params=pltpu.CompilerParams(needs_layout_passes=False),
    scratch_types=[
        pltpu.VMEM((EPW,), jnp.int32),
        pltpu.VMEM((N,), jnp.float32),
    ],
)
def _deg_kernel(col_hbm, out_hbm, cidx_v, deg_v):
    c = lax.axis_index("c")
    s = lax.axis_index("s")
    wid = c * NS + s
    pltpu.sync_copy(col_hbm.at[pl.ds(wid * EPW, EPW)], cidx_v)

    def zbody(i, carry):
        deg_v[pl.ds(i * L, L)] = jnp.zeros((L,), jnp.float32)
        return carry

    lax.fori_loop(0, N // L, zbody, 0)
    ones = jnp.ones((L,), jnp.float32)

    def body(i, carry):
        idx = cidx_v[pl.ds(i * L, L)]
        plsc.addupdate_scatter(deg_v, [idx], ones)
        return carry

    lax.fori_loop(0, EPW // L, body, 0)
    pltpu.sync_copy(deg_v, out_hbm.at[pl.ds(wid * N, N)])


NBUF = 6


@functools.partial(
    pl.kernel,
    out_type=jax.ShapeDtypeStruct((NC, N, D), jnp.float32),
    mesh=_mesh,
    scratch_types=[
        [pltpu.VMEM((CHK,), jnp.int32) for _ in range(NBUF)],
        [pltpu.VMEM((CHK,), jnp.int32) for _ in range(NBUF)],
        [pltpu.VMEM((CHK, D), jnp.float32) for _ in range(NBUF)],
        pltpu.VMEM((CHK_T,), jnp.int32),
        pltpu.VMEM((CHK_T,), jnp.int32),
        pltpu.VMEM_SHARED((N, D), jnp.float32),
        pltpu.SemaphoreType.DMA((NBUF,)),
        pltpu.SemaphoreType.DMA((NBUF,)),
        pltpu.SemaphoreType.DMA((NBUF,)),
    ],
)
def _agg_kernel(g_hbm, row_hbm, col_hbm, z_hbm, out_hbm,
                ridxs, cidxs, rows, ridx_t, cidx_t, acc_sh,
                isem, gsem, ssem):
    c = lax.axis_index("c")
    s = lax.axis_index("s")
    wid = c * NS + s

    def issue_idx(k, b):
        base = wid * EPW + k * CHK
        pltpu.async_copy(row_hbm.at[pl.ds(base, CHK)], ridxs[b], isem.at[b])
        pltpu.async_copy(col_hbm.at[pl.ds(base, CHK)], cidxs[b], isem.at[b])

    def wait_idx(b):
        pltpu.make_async_copy(
            row_hbm.at[pl.ds(0, CHK)], ridxs[b], isem.at[b]).wait()
        pltpu.make_async_copy(
            col_hbm.at[pl.ds(0, CHK)], cidxs[b], isem.at[b]).wait()

    def issue_gather(b):
        pltpu.async_copy(g_hbm.at[ridxs[b]], rows[b], gsem.at[b])

    def wait_gather(b):
        pltpu.make_async_copy(g_hbm.at[ridxs[b]], rows[b], gsem.at[b]).wait()

    def issue_scatter(b):
        pltpu.async_copy(rows[b], acc_sh.at[cidxs[b]], ssem.at[b], add=True)

    def wait_scatter(b):
        pltpu.make_async_copy(rows[b], acc_sh.at[cidxs[b]], ssem.at[b]).wait()

    _tile_rows(s, lambda sl, cnt: pltpu.sync_copy(
        z_hbm.at[pl.ds(0, cnt)], acc_sh.at[sl]))
    plsc.subcore_barrier()

    for b in range(3):
        issue_idx(b, b)
    for b in range(2):
        wait_idx(b)
        issue_gather(b)

    def outer(k0, carry):
        for b in range(NBUF):
            k = k0 * NBUF + b

            @pl.when(k >= 3)
            def _():
                wait_scatter((b + 3) % NBUF)

            @pl.when(k + 3 < NCHK)
            def _():
                issue_idx(k + 3, (b + 3) % NBUF)

            @pl.when(k + 2 < NCHK)
            def _():
                wait_idx((b + 2) % NBUF)
                issue_gather((b + 2) % NBUF)

            wait_gather(b)
            issue_scatter(b)
        return carry

    lax.fori_loop(0, NCHK // NBUF, outer, 0)
    for k in range(NCHK - 3, NCHK):
        wait_scatter(k % NBUF)
    base = wid * EPW + NCHK * CHK
    pltpu.async_copy(row_hbm.at[pl.ds(base, CHK_T)], ridx_t, isem.at[0])
    pltpu.async_copy(col_hbm.at[pl.ds(base, CHK_T)], cidx_t, isem.at[0])
    pltpu.make_async_copy(
        row_hbm.at[pl.ds(0, CHK_T)], ridx_t, isem.at[0]).wait()
    pltpu.make_async_copy(
        col_hbm.at[pl.ds(0, CHK_T)], cidx_t, isem.at[0]).wait()
    rows_t = rows[0].at[pl.ds(0, CHK_T)]
    pltpu.async_copy(g_hbm.at[ridx_t], rows_t, gsem.at[0])
    pltpu.make_async_copy(g_hbm.at[ridx_t], rows_t, gsem.at[0]).wait()
    pltpu.async_copy(rows_t, acc_sh.at[cidx_t], ssem.at[0], add=True)
    pltpu.make_async_copy(rows_t, acc_sh.at[cidx_t], ssem.at[0]).wait()
    plsc.subcore_barrier()
    _tile_rows(s, lambda sl, cnt: pltpu.sync_copy(
        acc_sh.at[sl], out_hbm.at[c, sl]))


@functools.partial(
    pl.kernel,
    out_type=jax.ShapeDtypeStruct((G, D), jnp.float32),
    mesh=_mesh,
    compiler_params=pltpu.CompilerParams(needs_layout_passes=False),
    scratch_types=[
        pltpu.VMEM((N,), jnp.int32),
        pltpu.VMEM((G,), jnp.int32),
        pltpu.VMEM((G, D), jnp.float32),
        pltpu.SemaphoreType.DMA,
    ],
)
def _final_kernel(h3_hbm, batch_hbm, out_hbm, batch_v, idx_v, rows_v, sem):
    c = lax.axis_index("c")
    s = lax.axis_index("s")
    wid = c * NS + s

    @pl.when(wid == 0)
    def _():
        pltpu.sync_copy(batch_hbm, batch_v)
        zero16 = jnp.zeros((L,), jnp.int32)
        for j in range(G // L):
            idx_v[pl.ds(j * L, L)] = zero16

        def body(i, carry):
            cnt, lastv = carry
            lane = lax.iota(jnp.int32, L)
            gpos = lane + i * L
            b = batch_v[pl.ds(i * L, L)]
            prev = b.at[jnp.maximum(lane - 1, 0)].get(mode="promise_in_bounds")
            prev = jnp.where(lane == 0, lastv, prev)
            mask = b != prev
            mi = mask.astype(jnp.int32)
            cum = plsc.cumsum(mi)
            pos = cnt + cum - mi
            wmask = mask & (pos < G)
            plsc.store_scatter(idx_v, [pos], gpos, mask=wmask)
            return cnt + jnp.sum(mi), jnp.max(b)

        lax.fori_loop(0, N // L, body, (0, -1))
        pltpu.async_copy(h3_hbm.at[idx_v], rows_v, sem).wait()
        pltpu.sync_copy(rows_v, out_hbm)


BR = 1000


def _dis_from(degp):
    deg = jnp.sum(degp, axis=1)
    return jnp.where(deg > 0, lax.rsqrt(deg), 0.0)


def _tc1_body(x_ref, w_ref, degp_ref, g_ref):
    dis = _dis_from(degp_ref[...])
    g_ref[...] = jnp.dot(x_ref[...] * dis[:, None], w_ref[...],
                         preferred_element_type=jnp.float32)


def _tc_mid_body(sp_ref, degp_ref, b_ref, w_ref, g_ref):
    dis = _dis_from(degp_ref[...])
    a = dis[:, None] * (sp_ref[0] + sp_ref[1]) + b_ref[...]
    a = jnp.where(a >= 0, a, 0.01 * a)
    g_ref[...] = jnp.dot(a * dis[:, None], w_ref[...],
                         preferred_element_type=jnp.float32)


def _tc_h3_body(sp_ref, degp_ref, b_ref, h_ref):
    dis = _dis_from(degp_ref[...])
    h_ref[...] = dis[:, None] * (sp_ref[0] + sp_ref[1]) + b_ref[...]


_degp_spec = pl.BlockSpec((BR, NW), lambda i: (i, 0))
_row_spec = pl.BlockSpec((BR, D), lambda i: (i, 0))
_w_spec = pl.BlockSpec((D, D), lambda i: (0, 0))
_b_spec = pl.BlockSpec((1, D), lambda i: (0, 0))
_sp_spec = pl.BlockSpec((NC, BR, D), lambda i: (0, i, 0))
_nd_shape = jax.ShapeDtypeStruct((N, D), jnp.float32)

_tc1 = pl.pallas_call(
    _tc1_body, grid=(N // BR,),
    in_specs=[_row_spec, _w_spec, _degp_spec],
    out_specs=_row_spec, out_shape=_nd_shape)

_tc_mid = pl.pallas_call(
    _tc_mid_body, grid=(N // BR,),
    in_specs=[_sp_spec, _degp_spec, _b_spec, _w_spec],
    out_specs=_row_spec, out_shape=_nd_shape)

_tc_h3 = pl.pallas_call(
    _tc_h3_body, grid=(N // BR,),
    in_specs=[_sp_spec, _degp_spec, _b_spec],
    out_specs=_row_spec, out_shape=_nd_shape)


def kernel(x, edge_index, batch, W1, b1, W2, b2, W3, b3):
    row = edge_index[0]
    col = edge_index[1]
    zD = jnp.zeros((RPT, D), jnp.float32)

    degp = _deg_kernel(col).reshape(NW, N).T
    g1 = _tc1(x, W1, degp)
    s1 = _agg_kernel(g1, row, col, zD)
    g2 = _tc_mid(s1, degp, b1.reshape(1, D), W2)
    s2 = _agg_kernel(g2, row, col, zD)
    g3 = _tc_mid(s2, degp, b2.reshape(1, D), W3)
    s3 = _agg_kernel(g3, row, col, zD)
    h3 = _tc_h3(s3, degp, b3.reshape(1, D))
    return _final_kernel(h3, batch)


# speedup vs baseline: 28.4301x; 1.1480x over previous
import functools

import jax
import jax.numpy as jnp
from jax import lax
from jax.experimental import pallas as pl
from jax.experimental.pallas import tpu as pltpu
from jax.experimental.pallas import tpu_sc as plsc

N = 10000
E = 320000
D = 128
G = 64

NC, NS, L = 2, 16, 16
NW = NC * NS
EPW = E // NW
CHK = 64
NCHK = EPW // CHK
CHK_T = EPW - NCHK * CHK
RPT = 632
RPT_LAST = N - (NS - 1) * RPT


def _tile_rows(s, io_fn):
    @pl.when(s < NS - 1)
    def _():
        io_fn(pl.ds(s * RPT, RPT), RPT)

    @pl.when(s == NS - 1)
    def _():
        io_fn(pl.ds((NS - 1) * RPT, RPT_LAST), RPT_LAST)

_mesh = plsc.VectorSubcoreMesh(
    core_axis_name="c", subcore_axis_name="s", num_cores=NC, num_subcores=NS)


@functools.partial(
    pl.kernel,
    out_type=jax.ShapeDtypeStruct((NW * N,), jnp.float32),
    mesh=_mesh,
    compiler_params=pltpu.CompilerParams(needs_layout_passes=False),
    scratch_types=[
        pltpu.VMEM((EPW,), jnp.int32),
        pltpu.VMEM((N,), jnp.float32),
    ],
)
def _deg_kernel(col_hbm, out_hbm, cidx_v, deg_v):
    c = lax.axis_index("c")
    s = lax.axis_index("s")
    wid = c * NS + s
    pltpu.sync_copy(col_hbm.at[pl.ds(wid * EPW, EPW)], cidx_v)

    def zbody(i, carry):
        deg_v[pl.ds(i * L, L)] = jnp.zeros((L,), jnp.float32)
        return carry

    lax.fori_loop(0, N // L, zbody, 0)
    ones = jnp.ones((L,), jnp.float32)

    def body(i, carry):
        idx = cidx_v[pl.ds(i * L, L)]
        plsc.addupdate_scatter(deg_v, [idx], ones)
        return carry

    lax.fori_loop(0, EPW // L, body, 0)
    pltpu.sync_copy(deg_v, out_hbm.at[pl.ds(wid * N, N)])


NBUF = 6


@functools.partial(
    pl.kernel,
    out_type=jax.ShapeDtypeStruct((NC, N, D), jnp.float32),
    mesh=_mesh,
    scratch_types=[
        [pltpu.VMEM((CHK,), jnp.int32) for _ in range(NBUF)],
        [pltpu.VMEM((CHK,), jnp.int32) for _ in range(NBUF)],
        [pltpu.VMEM((CHK, D), jnp.float32) for _ in range(NBUF)],
        pltpu.VMEM((CHK_T,), jnp.int32),
        pltpu.VMEM((CHK_T,), jnp.int32),
        pltpu.VMEM_SHARED((N, D), jnp.float32),
        pltpu.SemaphoreType.DMA((NBUF,)),
        pltpu.SemaphoreType.DMA((NBUF,)),
        pltpu.SemaphoreType.DMA((NBUF,)),
    ],
)
def _agg_kernel(g_hbm, row_hbm, col_hbm, z_hbm, out_hbm,
                ridxs, cidxs, rows, ridx_t, cidx_t, acc_sh,
                isem, gsem, ssem):
    c = lax.axis_index("c")
    s = lax.axis_index("s")
    wid = c * NS + s

    def issue_idx(k, b):
        base = wid * EPW + k * CHK
        pltpu.async_copy(row_hbm.at[pl.ds(base, CHK)], ridxs[b], isem.at[b])
        pltpu.async_copy(col_hbm.at[pl.ds(base, CHK)], cidxs[b], isem.at[b])

    def wait_idx(b):
        pltpu.make_async_copy(
            row_hbm.at[pl.ds(0, CHK)], ridxs[b], isem.at[b]).wait()
        pltpu.make_async_copy(
            col_hbm.at[pl.ds(0, CHK)], cidxs[b], isem.at[b]).wait()

    def issue_gather(b):
        pltpu.async_copy(g_hbm.at[ridxs[b]], rows[b], gsem.at[b])

    def wait_gather(b):
        pltpu.make_async_copy(g_hbm.at[ridxs[b]], rows[b], gsem.at[b]).wait()

    def issue_scatter(b):
        pltpu.async_copy(rows[b], acc_sh.at[cidxs[b]], ssem.at[b], add=True)

    def wait_scatter(b):
        pltpu.make_async_copy(rows[b], acc_sh.at[cidxs[b]], ssem.at[b]).wait()

    _tile_rows(s, lambda sl, cnt: pltpu.sync_copy(
        z_hbm.at[pl.ds(0, cnt)], acc_sh.at[sl]))
    plsc.subcore_barrier()

    for b in range(3):
        issue_idx(b, b)
    for b in range(2):
        wait_idx(b)
        issue_gather(b)

    def outer(k0, carry):
        for b in range(NBUF):
            k = k0 * NBUF + b

            @pl.when(k >= 3)
            def _():
                wait_scatter((b + 3) % NBUF)

            @pl.when(k + 3 < NCHK)
            def _():
                issue_idx(k + 3, (b + 3) % NBUF)

            @pl.when(k + 2 < NCHK)
            def _():
                wait_idx((b + 2) % NBUF)
                issue_gather((b + 2) % NBUF)

            wait_gather(b)
            issue_scatter(b)
        return carry

    lax.fori_loop(0, NCHK // NBUF, outer, 0)
    for k in range(NCHK - 3, NCHK):
        wait_scatter(k % NBUF)
    base = wid * EPW + NCHK * CHK
    pltpu.async_copy(row_hbm.at[pl.ds(base, CHK_T)], ridx_t, isem.at[0])
    pltpu.async_copy(col_hbm.at[pl.ds(base, CHK_T)], cidx_t, isem.at[0])
    pltpu.make_async_copy(
        row_hbm.at[pl.ds(0, CHK_T)], ridx_t, isem.at[0]).wait()
    pltpu.make_async_copy(
        col_hbm.at[pl.ds(0, CHK_T)], cidx_t, isem.at[0]).wait()
    rows_t = rows[0].at[pl.ds(0, CHK_T)]
    pltpu.async_copy(g_hbm.at[ridx_t], rows_t, gsem.at[0])
    pltpu.make_async_copy(g_hbm.at[ridx_t], rows_t, gsem.at[0]).wait()
    pltpu.async_copy(rows_t, acc_sh.at[cidx_t], ssem.at[0], add=True)
    pltpu.make_async_copy(rows_t, acc_sh.at[cidx_t], ssem.at[0]).wait()
    plsc.subcore_barrier()
    _tile_rows(s, lambda sl, cnt: pltpu.sync_copy(
        acc_sh.at[sl], out_hbm.at[c, sl]))


@functools.partial(
    pl.kernel,
    out_type=(jax.ShapeDtypeStruct((G,), jnp.int32),
              jax.ShapeDtypeStruct((L,), jnp.int32)),
    mesh=_mesh,
    compiler_params=pltpu.CompilerParams(needs_layout_passes=False),
    scratch_types=[
        pltpu.VMEM((N,), jnp.int32),
        pltpu.VMEM((G,), jnp.int32),
        pltpu.VMEM((L,), jnp.int32),
    ],
)
def _targets_kernel(batch_hbm, idx_hbm, cnt_hbm, batch_v, idx_v, cnt_v):
    c = lax.axis_index("c")
    s = lax.axis_index("s")
    wid = c * NS + s

    @pl.when(wid == 0)
    def _():
        pltpu.sync_copy(batch_hbm, batch_v)
        zero16 = jnp.zeros((L,), jnp.int32)
        for j in range(G // L):
            idx_v[pl.ds(j * L, L)] = zero16

        def body(i, carry):
            cnt, lastv = carry
            lane = lax.iota(jnp.int32, L)
            gpos = lane + i * L
            b = batch_v[pl.ds(i * L, L)]
            prev = b.at[jnp.maximum(lane - 1, 0)].get(mode="promise_in_bounds")
            prev = jnp.where(lane == 0, lastv, prev)
            mask = b != prev
            mi = mask.astype(jnp.int32)
            cum = plsc.cumsum(mi)
            pos = cnt + cum - mi
            wmask = mask & (pos < G)
            plsc.store_scatter(idx_v, [pos], gpos, mask=wmask)
            return cnt + jnp.sum(mi), jnp.max(b)

        cnt, _ = lax.fori_loop(0, N // L, body, (0, -1))
        cnt_v[...] = jnp.full((L,), cnt, jnp.int32)
        pltpu.sync_copy(idx_v, idx_hbm)
        pltpu.sync_copy(cnt_v, cnt_hbm)


EPT = E // NS
STRIP = 2000
NSTRIP = EPT // STRIP
MCAP = EPT + G
ACC_R = G + 8


@functools.partial(
    pl.kernel,
    out_type=jax.ShapeDtypeStruct((G, D), jnp.float32),
    mesh=_mesh,
    compiler_params=pltpu.CompilerParams(needs_layout_passes=False),
    scratch_types=[
        [pltpu.VMEM((STRIP,), jnp.int32) for _ in range(2)],
        [pltpu.VMEM((STRIP,), jnp.int32) for _ in range(2)],
        pltpu.VMEM((N,), jnp.int32),
        pltpu.VMEM((MCAP,), jnp.int32),
        pltpu.VMEM((MCAP,), jnp.int32),
        pltpu.VMEM((G,), jnp.int32),
        pltpu.VMEM((L,), jnp.int32),
        pltpu.VMEM((G,), jnp.int32),
        pltpu.VMEM((G,), jnp.int32),
        pltpu.VMEM((G, D), jnp.float32),
        pltpu.VMEM((G, D), jnp.float32),
        pltpu.VMEM((G, D), jnp.float32),
        pltpu.VMEM((D,), jnp.float32),
        pltpu.VMEM((G, D), jnp.float32),
        pltpu.VMEM_SHARED((ACC_R, D), jnp.float32),
        pltpu.SemaphoreType.DMA((2,)),
    ],
)
def _filt_kernel(g_hbm, row_hbm, col_hbm, tidx_hbm, tcnt_hbm, dispad_hbm,
                 b3_hbm, z_hbm, out_hbm,
                 rstrips, cstrips, slot_v, mrow_v, mslot_v, tidx_v, tcnt_v,
                 mrow64, mslot64, grows_v, accv, disv, b3v, hv, acc_sh, isem):
    c = lax.axis_index("c")
    s = lax.axis_index("s")
    lane = lax.iota(jnp.int32, L)

    def issue_strip(js, b):
        base = s * EPT + js * STRIP
        pltpu.async_copy(row_hbm.at[pl.ds(base, STRIP)], rstrips[b],
                         isem.at[b])
        pltpu.async_copy(col_hbm.at[pl.ds(base, STRIP)], cstrips[b],
                         isem.at[b])

    def wait_strip(b):
        pltpu.make_async_copy(
            row_hbm.at[pl.ds(0, STRIP)], rstrips[b], isem.at[b]).wait()
        pltpu.make_async_copy(
            col_hbm.at[pl.ds(0, STRIP)], cstrips[b], isem.at[b]).wait()

    @pl.when(c == 0)
    def _():
        @pl.when(s < ACC_R // 8)
        def _():
            pltpu.sync_copy(z_hbm.at[pl.ds(0, 8)], acc_sh.at[pl.ds(s * 8, 8)])

        pltpu.sync_copy(tidx_hbm, tidx_v)
        pltpu.sync_copy(tcnt_hbm, tcnt_v)
        issue_strip(0, 0)
        issue_strip(1, 1)
        cnt = tcnt_v[pl.ds(0, L)][0]

        def fbody(i, carry):
            slot_v[pl.ds(i * L, L)] = jnp.full((L,), -1, jnp.int32)
            return carry

        lax.fori_loop(0, N // L, fbody, 0)
        for j in range(G // L):
            tv = tidx_v[pl.ds(j * L, L)]
            gv = lane + j * L
            plsc.store_scatter(slot_v, [tv], gv, mask=gv < cnt)
        plsc.subcore_barrier()

        def scan_strip(b, mcnt):
            def sbody(i, mc):
                cv = cstrips[b][pl.ds(i * L, L)]
                sv = plsc.load_gather(slot_v, [cv])
                m = sv >= 0
                mi = m.astype(jnp.int32)
                cum = plsc.cumsum(mi)
                pos = mc + cum - mi
                rv = rstrips[b][pl.ds(i * L, L)]
                plsc.store_scatter(mrow_v, [pos], rv, mask=m)
                plsc.store_scatter(mslot_v, [pos], sv, mask=m)
                return mc + jnp.sum(mi)

            return lax.fori_loop(0, STRIP // L, sbody, mcnt)

        def obody(j2, mcnt):
            for b in range(2):
                js = j2 * 2 + b
                wait_strip(b)
                mcnt = scan_strip(b, mcnt)

                @pl.when(js + 2 < NSTRIP)
                def _():
                    issue_strip(js + 2, b)
            return mcnt

        mcnt = lax.fori_loop(0, NSTRIP // 2, obody, 0)

        padlen = (-mcnt) % G
        for j in range(G // L):
            pv = lane + j * L
            pm = pv < padlen
            plsc.store_scatter(mrow_v, [mcnt + pv],
                               jnp.zeros((L,), jnp.int32), mask=pm)
            plsc.store_scatter(mslot_v, [mcnt + pv],
                               jnp.full((L,), G, jnp.int32), mask=pm)

        def cbody(j, carry):
            for t in range(G // L):
                off = j * G + t * L
                mrow64[pl.ds(t * L, L)] = mrow_v[pl.ds(off, L)]
                mslot64[pl.ds(t * L, L)] = mslot_v[pl.ds(off, L)]
            pltpu.sync_copy(g_hbm.at[mrow64], grows_v)
            pltpu.sync_copy(grows_v, acc_sh.at[mslot64], add=True)
            return carry

        lax.fori_loop(0, (mcnt + G - 1) // G, cbody, 0)
        plsc.subcore_barrier()

        @pl.when(s == 0)
        def _():
            pltpu.sync_copy(acc_sh.at[pl.ds(0, G)], accv)
            pltpu.sync_copy(dispad_hbm.at[tidx_v], disv)
            pltpu.sync_copy(b3_hbm, b3v)
            cnt2 = tcnt_v[pl.ds(0, L)][0]
            a0 = [accv[0, pl.ds(j * L, L)] for j in range(D // L)]
            for t in range(G):
                tin = t < cnt2
                for j in range(D // L):
                    a = jnp.where(tin, accv[t, pl.ds(j * L, L)], a0[j])
                    h = a * disv[t, pl.ds(j * L, L)] + b3v[pl.ds(j * L, L)]
                    hv[t, pl.ds(j * L, L)] = h
            pltpu.sync_copy(hv, out_hbm)


BR = 1000


def _dis_from(degp):
    deg = jnp.sum(degp, axis=1)
    return jnp.where(deg > 0, lax.rsqrt(deg), 0.0)


def _tc1_body(x_ref, w_ref, degp_ref, g_ref, dis_ref):
    dis = _dis_from(degp_ref[...])
    g_ref[...] = jnp.dot(x_ref[...] * dis[:, None], w_ref[...],
                         preferred_element_type=jnp.float32)
    dis_ref[...] = jnp.broadcast_to(dis[:, None], (BR, D))


def _tc_mid_body(sp_ref, degp_ref, b_ref, w_ref, g_ref):
    dis = _dis_from(degp_ref[...])
    a = dis[:, None] * (sp_ref[0] + sp_ref[1]) + b_ref[...]
    a = jnp.where(a >= 0, a, 0.01 * a)
    g_ref[...] = jnp.dot(a * dis[:, None], w_ref[...],
                         preferred_element_type=jnp.float32)


_degp_spec = pl.BlockSpec((BR, NW), lambda i: (i, 0))
_row_spec = pl.BlockSpec((BR, D), lambda i: (i, 0))
_w_spec = pl.BlockSpec((D, D), lambda i: (0, 0))
_b_spec = pl.BlockSpec((1, D), lambda i: (0, 0))
_sp_spec = pl.BlockSpec((NC, BR, D), lambda i: (0, i, 0))
_nd_shape = jax.ShapeDtypeStruct((N, D), jnp.float32)

_tc1 = pl.pallas_call(
    _tc1_body, grid=(N // BR,),
    in_specs=[_row_spec, _w_spec, _degp_spec],
    out_specs=[_row_spec, _row_spec], out_shape=[_nd_shape, _nd_shape])

_tc_mid = pl.pallas_call(
    _tc_mid_body, grid=(N // BR,),
    in_specs=[_sp_spec, _degp_spec, _b_spec, _w_spec],
    out_specs=_row_spec, out_shape=_nd_shape)


def kernel(x, edge_index, batch, W1, b1, W2, b2, W3, b3):
    row = edge_index[0]
    col = edge_index[1]
    zD = jnp.zeros((RPT, D), jnp.float32)

    tidx, tcnt = _targets_kernel(batch)
    degp = _deg_kernel(col).reshape(NW, N).T
    g1, dispad = _tc1(x, W1, degp)
    s1 = _agg_kernel(g1, row, col, zD)
    g2 = _tc_mid(s1, degp, b1.reshape(1, D), W2)
    s2 = _agg_kernel(g2, row, col, zD)
    g3 = _tc_mid(s2, degp, b2.reshape(1, D), W3)
    return _filt_kernel(g3, row, col, tidx, tcnt, dispad, b3, zD)
